# baseline (speedup 1.0000x reference)
import functools

import jax
import jax.numpy as jnp
from jax import lax
from jax.experimental import pallas as pl
from jax.experimental.pallas import tpu as pltpu
from jax.experimental.pallas import tpu_sc as plsc

EMB = 32
B = 4096
L = 200
N = B * L
T4 = 250000
NW = 32
BW = B // NW
LB = 2
WORDS = LB * BW
NBLK = L // LB
NBUF = 2
NOUTER = NBLK // NBUF


def _make_gather():
    mesh = plsc.VectorSubcoreMesh(core_axis_name="c", subcore_axis_name="s")

    scratch = (
        [pltpu.VMEM((LB, BW), jnp.int32) for _ in range(NBUF)]
        + [pltpu.VMEM((WORDS,), jnp.int32) for _ in range(NBUF)]
        + [pltpu.VMEM((WORDS, 128), jnp.float32) for _ in range(NBUF)]
        + [pltpu.VMEM((LB, EMB, BW), jnp.float32) for _ in range(NBUF)]
        + [pltpu.SemaphoreType.DMA for _ in range(2 * NBUF)]
    )

    @functools.partial(
        pl.kernel,
        mesh=mesh,
        out_type=jax.ShapeDtypeStruct((L, EMB, B), jnp.float32),
        scratch_types=scratch,
        compiler_params=pltpu.CompilerParams(needs_layout_passes=False),
    )
    def gather_kernel(ids_hbm, table_hbm, out_hbm, *scratch_refs):
        ids_v = scratch_refs[:NBUF]
        line_v = scratch_refs[NBUF:2 * NBUF]
        lines_v = scratch_refs[2 * NBUF:3 * NBUF]
        out_v = scratch_refs[3 * NBUF:4 * NBUF]
        gsem = scratch_refs[4 * NBUF:5 * NBUF]
        wsem = scratch_refs[5 * NBUF:6 * NBUF]

        wid = lax.axis_index("s") * 2 + lax.axis_index("c")
        boff = wid * BW
        iota = lax.iota(jnp.int32, 16)

        def stage_and_fire(b, u):
            pltpu.sync_copy(
                ids_hbm.at[pl.ds(u * LB, LB), pl.ds(boff, BW)], ids_v[b])

            def lines_body(t, carry):
                iv = ids_v[b][t >> 3, pl.ds((t & 7) * 16, 16)]
                line_v[b][pl.ds(t * 16, 16)] = lax.shift_right_logical(iv, 2)
                return carry

            lax.fori_loop(0, WORDS // 16, lines_body, 0)
            pltpu.async_copy(table_hbm.at[line_v[b]], lines_v[b], gsem[b])

        def gather_wait(b):
            pltpu.make_async_copy(
                table_hbm.at[line_v[b]], lines_v[b], gsem[b]).wait()

        def extract(b):
            def q_body(q, carry):
                lp = q >> 3
                kg = q & 7
                iv = ids_v[b][lp, pl.ds(kg * 16, 16)]
                sub32 = lax.shift_left(iv & 3, 5)
                rows = lp * BW + kg * 16 + iota
                for e in range(EMB):
                    val = plsc.load_gather(lines_v[b], [rows, sub32 + e])
                    out_v[b][lp, e, pl.ds(kg * 16, 16)] = val
                return carry

            lax.fori_loop(0, LB * (BW // 16), q_body, 0)

        def write_start(b, u):
            pltpu.async_copy(
                out_v[b],
                out_hbm.at[pl.ds(u * LB, LB), :, pl.ds(boff, BW)],
                wsem[b])

        def write_wait(b, u):
            pltpu.make_async_copy(
                out_v[b],
                out_hbm.at[pl.ds(u * LB, LB), :, pl.ds(boff, BW)],
                wsem[b]).wait()

        for b in range(NBUF):
            stage_and_fire(b, b)
        for b in range(NBUF):
            gather_wait(b)
            extract(b)
            write_start(b, b)
            stage_and_fire(b, b + NBUF)

        def outer_body(g, carry):
            for b in range(NBUF):
                u = g * NBUF + b
                gather_wait(b)
                write_wait(b, u - NBUF)
                extract(b)
                write_start(b, u)
                stage_and_fire(b, u + NBUF)
            return carry

        lax.fori_loop(1, NOUTER - 1, outer_body, 0)

        last = (NOUTER - 1) * NBUF
        for b in range(NBUF):
            gather_wait(b)
            write_wait(b, last + b - NBUF)
            extract(b)
            write_start(b, last + b)
        for b in range(NBUF):
            write_wait(b, last + b)

    return gather_kernel


_gather_r4 = _make_gather()


def kernel(word_ids, table):
    ids_t = word_ids.T
    table4 = table.reshape(T4, 4 * EMB)
    out = _gather_r4(ids_t, table4)
    return out.transpose(2, 0, 1)


# speedup vs baseline: 1.2795x; 1.2795x over previous
import functools

import jax
import jax.numpy as jnp
from jax import lax
from jax.experimental import pallas as pl
from jax.experimental.pallas import tpu as pltpu
from jax.experimental.pallas import tpu_sc as plsc

EMB = 32
B = 4096
L = 200
N = B * L
T4 = 250000
NW = 32
BW = B // NW
LB = 2
WORDS = LB * BW
NBLK = L // LB
NBUF = 2
NOUTER = NBLK // NBUF


def _make_gather():
    mesh = plsc.VectorSubcoreMesh(core_axis_name="c", subcore_axis_name="s")

    scratch = (
        [pltpu.VMEM((LB, BW), jnp.int32) for _ in range(NBUF)]
        + [pltpu.VMEM((WORDS,), jnp.int32) for _ in range(NBUF)]
        + [pltpu.VMEM((WORDS, 128), jnp.float32) for _ in range(NBUF)]
        + [pltpu.VMEM((LB, EMB, BW), jnp.float32) for _ in range(NBUF)]
        + [pltpu.SemaphoreType.DMA for _ in range(2 * NBUF)]
    )

    @functools.partial(
        pl.kernel,
        mesh=mesh,
        out_type=jax.ShapeDtypeStruct((L, EMB, B), jnp.float32),
        scratch_types=scratch,
        compiler_params=pltpu.CompilerParams(needs_layout_passes=False),
    )
    def gather_kernel(ids_hbm, table_hbm, out_hbm, *scratch_refs):
        ids_v = scratch_refs[:NBUF]
        line_v = scratch_refs[NBUF:2 * NBUF]
        lines_v = scratch_refs[2 * NBUF:3 * NBUF]
        out_v = scratch_refs[3 * NBUF:4 * NBUF]
        gsem = scratch_refs[4 * NBUF:5 * NBUF]
        wsem = scratch_refs[5 * NBUF:6 * NBUF]

        wid = lax.axis_index("s") * 2 + lax.axis_index("c")
        boff = wid * BW
        iota = lax.iota(jnp.int32, 16)

        def stage_and_fire(b, u):
            pltpu.sync_copy(
                ids_hbm.at[pl.ds(u * LB, LB), pl.ds(boff, BW)], ids_v[b])

            def lines_body(t, carry):
                iv = ids_v[b][t >> 3, pl.ds((t & 7) * 16, 16)]
                line_v[b][pl.ds(t * 16, 16)] = lax.shift_right_logical(iv, 2)
                return carry

            lax.fori_loop(0, WORDS // 16, lines_body, 0)
            pltpu.async_copy(table_hbm.at[line_v[b]], lines_v[b], gsem[b])

        def gather_wait(b):
            pltpu.make_async_copy(
                table_hbm.at[line_v[b]], lines_v[b], gsem[b]).wait()

        def extract(b):
            def q_body(q, carry):
                lp = q >> 3
                kg = q & 7
                iv = ids_v[b][lp, pl.ds(kg * 16, 16)]
                sub32 = lax.shift_left(iv & 3, 5)
                rows = lp * BW + kg * 16 + iota
                for eh in range(0, EMB, 16):
                    vals = [
                        plsc.load_gather(lines_v[b], [rows, sub32 + (eh + e)])
                        for e in range(16)
                    ]
                    for e in range(16):
                        out_v[b][lp, eh + e, pl.ds(kg * 16, 16)] = vals[e]
                return carry

            lax.fori_loop(0, LB * (BW // 16), q_body, 0)

        def write_start(b, u):
            pltpu.async_copy(
                out_v[b],
                out_hbm.at[pl.ds(u * LB, LB), :, pl.ds(boff, BW)],
                wsem[b])

        def write_wait(b, u):
            pltpu.make_async_copy(
                out_v[b],
                out_hbm.at[pl.ds(u * LB, LB), :, pl.ds(boff, BW)],
                wsem[b]).wait()

        for b in range(NBUF):
            stage_and_fire(b, b)
        for b in range(NBUF):
            gather_wait(b)
            extract(b)
            write_start(b, b)
            stage_and_fire(b, b + NBUF)

        def outer_body(g, carry):
            for b in range(NBUF):
                u = g * NBUF + b
                gather_wait(b)
                write_wait(b, u - NBUF)
                extract(b)
                write_start(b, u)
                stage_and_fire(b, u + NBUF)
            return carry

        lax.fori_loop(1, NOUTER - 1, outer_body, 0)

        last = (NOUTER - 1) * NBUF
        for b in range(NBUF):
            gather_wait(b)
            write_wait(b, last + b - NBUF)
            extract(b)
            write_start(b, last + b)
        for b in range(NBUF):
            write_wait(b, last + b)

    return gather_kernel


_gather_r4 = _make_gather()


def kernel(word_ids, table):
    ids_t = word_ids.T
    table4 = table.reshape(T4, 4 * EMB)
    out = _gather_r4(ids_t, table4)
    return out.transpose(2, 0, 1)
